# Initial kernel scaffold; baseline (speedup 1.0000x reference)
import jax
import jax.numpy as jnp
from jax.experimental import pallas as pl


def kernel(x, edge_index, W1, b1, W2, b2):
    raise NotImplementedError("write your pallas kernel here")


# speedup vs baseline: 29.2433x; 29.2433x over previous
import functools

import jax
import jax.numpy as jnp
from jax import lax
from jax.experimental import pallas as pl
from jax.experimental.pallas import tpu as pltpu
from jax.experimental.pallas import tpu_sc as plsc

N_NODES = 10000
N_EDGES = 320000
D_FEAT = 128
D_HID = 16
N_CLS = 16

NC = 2
NS = 16
NW = NC * NS
G = 128

N_PAD = 10240
ROWS_PER_TILE = N_PAD // NS
EPW = 10240
K = EPW // G
E_PAD = EPW * NW

def _worker_ids():
    cid = lax.axis_index("c")
    sid = lax.axis_index("s")
    return cid, sid, sid * NC + cid


@functools.cache
def _sc_kernels():
    mesh = plsc.VectorSubcoreMesh(core_axis_name="c", subcore_axis_name="s")
    params = pltpu.CompilerParams(use_tc_tiling_on_sc=False)

    @functools.partial(
        pl.kernel,
        mesh=mesh,
        compiler_params=params,
        out_type=jax.ShapeDtypeStruct((NC, N_PAD, 16), jnp.float32),
        scratch_types=[
            pltpu.VMEM((K, G), jnp.int32),
            pltpu.VMEM((G, 16), jnp.float32),
            pltpu.VMEM_SHARED((N_PAD, 16), jnp.float32),
        ],
    )
    def deg_kernel(dstp, ones_hbm, zeros_hbm, out, dst_v, ones_v, acc):
        cid, sid, wid = _worker_ids()
        pltpu.sync_copy(dstp.at[wid], dst_v)
        pltpu.sync_copy(ones_hbm, ones_v)
        sl = pl.ds(sid * ROWS_PER_TILE, ROWS_PER_TILE)
        pltpu.sync_copy(zeros_hbm.at[sl], acc.at[sl])
        plsc.subcore_barrier()

        def body(j, carry):
            pltpu.sync_copy(ones_v, acc.at[dst_v.at[j]], add=True)
            return carry

        lax.fori_loop(0, K, body, 0)
        plsc.subcore_barrier()
        pltpu.sync_copy(acc.at[sl], out.at[cid, sl])

    @functools.partial(
        pl.kernel,
        mesh=mesh,
        compiler_params=params,
        out_type=jax.ShapeDtypeStruct((NC, N_PAD, 16), jnp.float32),
        scratch_types=[
            pltpu.VMEM((K, G), jnp.int32),
            pltpu.VMEM((K, G), jnp.int32),
            pltpu.VMEM((G, 16), jnp.float32),
            pltpu.VMEM_SHARED((N_PAD, 16), jnp.float32),
            pltpu.SemaphoreType.DMA,
        ],
    )
    def agg_kernel(hp, srcp, dstp, zeros_hbm, out, src_v, dst_v, rows_v, acc, sem):
        cid, sid, wid = _worker_ids()
        pltpu.sync_copy(srcp.at[wid], src_v)
        pltpu.sync_copy(dstp.at[wid], dst_v)
        sl = pl.ds(sid * ROWS_PER_TILE, ROWS_PER_TILE)
        pltpu.sync_copy(zeros_hbm.at[sl], acc.at[sl])
        plsc.subcore_barrier()

        def body(j, carry):
            pltpu.async_copy(hp.at[src_v.at[j]], rows_v, sem).wait()
            pltpu.sync_copy(rows_v, acc.at[dst_v.at[j]], add=True)
            return carry

        lax.fori_loop(0, K, body, 0)
        plsc.subcore_barrier()
        pltpu.sync_copy(acc.at[sl], out.at[cid, sl])

    return deg_kernel, agg_kernel


def _t2_body(degp_ref, x_ref, w1_ref, hp1_ref, self1_ref, dinv_ref):
    deg = degp_ref[0, :N_NODES, 0] + degp_ref[1, :N_NODES, 0] + 1.0
    dinv = lax.rsqrt(jnp.maximum(deg, 1.0))[:, None]
    h1 = jnp.dot(x_ref[...], w1_ref[...], preferred_element_type=jnp.float32)
    hp1_ref[...] = dinv * h1
    self1_ref[...] = dinv * dinv * h1
    dinv_ref[...] = jnp.broadcast_to(dinv, (N_NODES, 16))


def _t3_body(aggp_ref, dinv_ref, self1_ref, b1_ref, w2_ref, hp2_ref, self2_ref):
    agg = aggp_ref[0, :N_NODES, :] + aggp_ref[1, :N_NODES, :]
    dinv = dinv_ref[...]
    x2 = jnp.maximum(dinv * agg + self1_ref[...] + b1_ref[...], 0.0)
    h2 = jnp.dot(x2, w2_ref[...], preferred_element_type=jnp.float32)
    hp2_ref[...] = dinv * h2
    self2_ref[...] = dinv * dinv * h2


def _t4_body(aggp_ref, dinv_ref, self2_ref, b2_ref, out_ref):
    agg = aggp_ref[0, :N_NODES, :] + aggp_ref[1, :N_NODES, :]
    out_ref[...] = dinv_ref[...] * agg + self2_ref[...] + b2_ref[...]


_t2 = pl.pallas_call(
    _t2_body,
    out_shape=(
        jax.ShapeDtypeStruct((N_NODES, D_HID), jnp.float32),
        jax.ShapeDtypeStruct((N_NODES, D_HID), jnp.float32),
        jax.ShapeDtypeStruct((N_NODES, 16), jnp.float32),
    ),
)

_t3 = pl.pallas_call(
    _t3_body,
    out_shape=(
        jax.ShapeDtypeStruct((N_NODES, N_CLS), jnp.float32),
        jax.ShapeDtypeStruct((N_NODES, N_CLS), jnp.float32),
    ),
)

_t4 = pl.pallas_call(
    _t4_body,
    out_shape=jax.ShapeDtypeStruct((N_NODES, N_CLS), jnp.float32),
)


def kernel(x, edge_index, W1, b1, W2, b2):
    src = edge_index[0].astype(jnp.int32)
    dst = edge_index[1].astype(jnp.int32)
    npad = E_PAD - N_EDGES
    srcp = jnp.concatenate([src, jnp.zeros((npad,), jnp.int32)])
    dstp = jnp.concatenate([dst, jnp.full((npad,), N_NODES, jnp.int32)])
    srcp = srcp.reshape(NW, K, G)
    dstp = dstp.reshape(NW, K, G)

    ones = jnp.ones((G, 16), jnp.float32)
    zeros = jnp.zeros((N_PAD, 16), jnp.float32)

    deg_kernel, agg_kernel = _sc_kernels()
    degp = deg_kernel(dstp, ones, zeros)
    hp1, self1, dinv = _t2(degp, x, W1)
    agg1 = agg_kernel(hp1, srcp, dstp, zeros)
    hp2, self2 = _t3(agg1, dinv, self1, b1.reshape(1, 16), W2)
    agg2 = agg_kernel(hp2, srcp, dstp, zeros)
    return _t4(agg2, dinv, self2, b2.reshape(1, 16))


# speedup vs baseline: 37.9054x; 1.2962x over previous
import functools

import jax
import jax.numpy as jnp
from jax import lax
from jax.experimental import pallas as pl
from jax.experimental.pallas import tpu as pltpu
from jax.experimental.pallas import tpu_sc as plsc

N_NODES = 10000
N_EDGES = 320000
D_FEAT = 128
D_HID = 16
N_CLS = 16

NC = 2
NS = 16
NW = NC * NS
G = 128

N_PAD = 10240
ROWS_PER_TILE = N_PAD // NS
EPW = 10240
K = EPW // G
E_PAD = EPW * NW

def _worker_ids():
    cid = lax.axis_index("c")
    sid = lax.axis_index("s")
    return cid, sid, sid * NC + cid


@functools.cache
def _sc_kernels():
    mesh = plsc.VectorSubcoreMesh(core_axis_name="c", subcore_axis_name="s")
    params = pltpu.CompilerParams(use_tc_tiling_on_sc=False)

    @functools.partial(
        pl.kernel,
        mesh=mesh,
        compiler_params=params,
        out_type=jax.ShapeDtypeStruct((NC, N_PAD, 16), jnp.float32),
        scratch_types=[
            pltpu.VMEM((K, G), jnp.int32),
            pltpu.VMEM((G, 16), jnp.float32),
            pltpu.VMEM_SHARED((N_PAD, 16), jnp.float32),
        ],
    )
    def deg_kernel(dstp, ones_hbm, zeros_hbm, out, dst_v, ones_v, acc):
        cid, sid, wid = _worker_ids()
        pltpu.sync_copy(dstp.at[wid], dst_v)
        pltpu.sync_copy(ones_hbm, ones_v)
        sl = pl.ds(sid * ROWS_PER_TILE, ROWS_PER_TILE)
        pltpu.sync_copy(zeros_hbm.at[sl], acc.at[sl])
        plsc.subcore_barrier()

        def body(j, carry):
            pltpu.sync_copy(ones_v, acc.at[dst_v.at[j]], add=True)
            return carry

        lax.fori_loop(0, K, body, 0)
        plsc.subcore_barrier()
        pltpu.sync_copy(acc.at[sl], out.at[cid, sl])

    @functools.partial(
        pl.kernel,
        mesh=mesh,
        compiler_params=params,
        out_type=jax.ShapeDtypeStruct((NC, N_PAD, 16), jnp.float32),
        scratch_types=[
            pltpu.VMEM((K, G), jnp.int32),
            pltpu.VMEM((K, G), jnp.int32),
            pltpu.VMEM((G, 16), jnp.float32),
            pltpu.VMEM((G, 16), jnp.float32),
            pltpu.VMEM_SHARED((N_PAD, 16), jnp.float32),
            pltpu.SemaphoreType.DMA,
            pltpu.SemaphoreType.DMA,
        ],
    )
    def agg_kernel(hp, srcp, dstp, zeros_hbm, out,
                   src_v, dst_v, rows0, rows1, acc, sem0, sem1):
        cid, sid, wid = _worker_ids()
        pltpu.sync_copy(srcp.at[wid], src_v)
        pltpu.sync_copy(dstp.at[wid], dst_v)
        sl = pl.ds(sid * ROWS_PER_TILE, ROWS_PER_TILE)
        pltpu.sync_copy(zeros_hbm.at[sl], acc.at[sl])
        plsc.subcore_barrier()

        rows = (rows0, rows1)
        sems = (sem0, sem1)
        pltpu.async_copy(hp.at[src_v.at[0]], rows0, sem0)

        def grp(g, carry):
            j0 = g * 2
            for b in range(2):
                j = j0 + b
                jn = j + 1

                @pl.when(jn < K)
                def _prefetch():
                    pltpu.async_copy(hp.at[src_v.at[jn]], rows[1 - b], sems[1 - b])

                pltpu.make_async_copy(hp.at[pl.ds(0, G)], rows[b], sems[b]).wait()
                pltpu.sync_copy(rows[b], acc.at[dst_v.at[j]], add=True)
            return carry

        lax.fori_loop(0, K // 2, grp, 0)
        plsc.subcore_barrier()
        pltpu.sync_copy(acc.at[sl], out.at[cid, sl])

    return deg_kernel, agg_kernel


def _t2_body(degp_ref, x_ref, w1_ref, hp1_ref, self1_ref, dinv_ref):
    deg = degp_ref[0, :N_NODES, 0] + degp_ref[1, :N_NODES, 0] + 1.0
    dinv = lax.rsqrt(jnp.maximum(deg, 1.0))[:, None]
    h1 = jnp.dot(x_ref[...], w1_ref[...], preferred_element_type=jnp.float32)
    hp1_ref[...] = dinv * h1
    self1_ref[...] = dinv * dinv * h1
    dinv_ref[...] = jnp.broadcast_to(dinv, (N_NODES, 16))


def _t3_body(aggp_ref, dinv_ref, self1_ref, b1_ref, w2_ref, hp2_ref, self2_ref):
    agg = aggp_ref[0, :N_NODES, :] + aggp_ref[1, :N_NODES, :]
    dinv = dinv_ref[...]
    x2 = jnp.maximum(dinv * agg + self1_ref[...] + b1_ref[...], 0.0)
    h2 = jnp.dot(x2, w2_ref[...], preferred_element_type=jnp.float32)
    hp2_ref[...] = dinv * h2
    self2_ref[...] = dinv * dinv * h2


def _t4_body(aggp_ref, dinv_ref, self2_ref, b2_ref, out_ref):
    agg = aggp_ref[0, :N_NODES, :] + aggp_ref[1, :N_NODES, :]
    out_ref[...] = dinv_ref[...] * agg + self2_ref[...] + b2_ref[...]


_t2 = pl.pallas_call(
    _t2_body,
    out_shape=(
        jax.ShapeDtypeStruct((N_NODES, D_HID), jnp.float32),
        jax.ShapeDtypeStruct((N_NODES, D_HID), jnp.float32),
        jax.ShapeDtypeStruct((N_NODES, 16), jnp.float32),
    ),
)

_t3 = pl.pallas_call(
    _t3_body,
    out_shape=(
        jax.ShapeDtypeStruct((N_NODES, N_CLS), jnp.float32),
        jax.ShapeDtypeStruct((N_NODES, N_CLS), jnp.float32),
    ),
)

_t4 = pl.pallas_call(
    _t4_body,
    out_shape=jax.ShapeDtypeStruct((N_NODES, N_CLS), jnp.float32),
)


def kernel(x, edge_index, W1, b1, W2, b2):
    src = edge_index[0].astype(jnp.int32)
    dst = edge_index[1].astype(jnp.int32)
    npad = E_PAD - N_EDGES
    srcp = jnp.concatenate([src, jnp.zeros((npad,), jnp.int32)])
    dstp = jnp.concatenate([dst, jnp.full((npad,), N_NODES, jnp.int32)])
    srcp = srcp.reshape(NW, K, G)
    dstp = dstp.reshape(NW, K, G)

    ones = jnp.ones((G, 16), jnp.float32)
    zeros = jnp.zeros((N_PAD, 16), jnp.float32)

    deg_kernel, agg_kernel = _sc_kernels()
    degp = deg_kernel(dstp, ones, zeros)
    hp1, self1, dinv = _t2(degp, x, W1)
    agg1 = agg_kernel(hp1, srcp, dstp, zeros)
    hp2, self2 = _t3(agg1, dinv, self1, b1.reshape(1, 16), W2)
    agg2 = agg_kernel(hp2, srcp, dstp, zeros)
    return _t4(agg2, dinv, self2, b2.reshape(1, 16))


# speedup vs baseline: 47.3460x; 1.2491x over previous
import functools

import jax
import jax.numpy as jnp
from jax import lax
from jax.experimental import pallas as pl
from jax.experimental.pallas import tpu as pltpu
from jax.experimental.pallas import tpu_sc as plsc

N_NODES = 10000
N_EDGES = 320000
D_FEAT = 128
D_HID = 16
N_CLS = 16

NC = 2
NS = 16
NW = NC * NS
G = 128

N_PAD = 10240
ROWS_PER_TILE = N_PAD // NS
EPW = 10240
K = EPW // G
E_PAD = EPW * NW

def _worker_ids():
    cid = lax.axis_index("c")
    sid = lax.axis_index("s")
    return cid, sid, sid * NC + cid


@functools.cache
def _sc_kernels():
    mesh = plsc.VectorSubcoreMesh(core_axis_name="c", subcore_axis_name="s")
    params = pltpu.CompilerParams(use_tc_tiling_on_sc=False)

    @functools.partial(
        pl.kernel,
        mesh=mesh,
        compiler_params=params,
        out_type=jax.ShapeDtypeStruct((NC, N_PAD, 16), jnp.float32),
        scratch_types=[
            pltpu.VMEM((K, G), jnp.int32),
            pltpu.VMEM((G, 16), jnp.float32),
            pltpu.VMEM_SHARED((N_PAD, 16), jnp.float32),
        ],
    )
    def deg_kernel(dstp, ones_hbm, zeros_hbm, out, dst_v, ones_v, acc):
        cid, sid, wid = _worker_ids()
        pltpu.sync_copy(dstp.at[wid], dst_v)
        pltpu.sync_copy(ones_hbm, ones_v)
        sl = pl.ds(sid * ROWS_PER_TILE, ROWS_PER_TILE)
        pltpu.sync_copy(zeros_hbm.at[sl], acc.at[sl])
        plsc.subcore_barrier()

        def body(j, carry):
            pltpu.sync_copy(ones_v, acc.at[dst_v.at[j]], add=True)
            return carry

        lax.fori_loop(0, K, body, 0)
        plsc.subcore_barrier()
        pltpu.sync_copy(acc.at[sl], out.at[cid, sl])

    @functools.partial(
        pl.kernel,
        mesh=mesh,
        compiler_params=params,
        out_type=jax.ShapeDtypeStruct((NC, N_PAD, 16), jnp.float32),
        scratch_types=[
            pltpu.VMEM((K, G), jnp.int32),
            pltpu.VMEM((K, G), jnp.int32),
            pltpu.VMEM((G, 16), jnp.float32),
            pltpu.VMEM((G, 16), jnp.float32),
            pltpu.VMEM_SHARED((N_PAD, 16), jnp.float32),
            pltpu.SemaphoreType.DMA,
            pltpu.SemaphoreType.DMA,
        ],
    )
    def agg_kernel(hp, srcp, dstp, zeros_hbm, out,
                   src_v, dst_v, rows0, rows1, acc, sem0, sem1):
        cid, sid, wid = _worker_ids()
        pltpu.sync_copy(srcp.at[wid], src_v)
        pltpu.sync_copy(dstp.at[wid], dst_v)
        sl = pl.ds(sid * ROWS_PER_TILE, ROWS_PER_TILE)
        pltpu.sync_copy(zeros_hbm.at[sl], acc.at[sl])
        plsc.subcore_barrier()

        rows = (rows0, rows1)
        sems = (sem0, sem1)
        pltpu.async_copy(hp.at[src_v.at[0]], rows0, sem0)

        def grp(g, carry):
            j0 = g * 2
            for b in range(2):
                j = j0 + b
                jn = j + 1

                @pl.when(jn < K)
                def _prefetch():
                    pltpu.async_copy(hp.at[src_v.at[jn]], rows[1 - b], sems[1 - b])

                pltpu.make_async_copy(hp.at[pl.ds(0, G)], rows[b], sems[b]).wait()
                pltpu.sync_copy(rows[b], acc.at[dst_v.at[j]], add=True)
            return carry

        lax.fori_loop(0, K // 2, grp, 0)
        plsc.subcore_barrier()
        pltpu.sync_copy(acc.at[sl], out.at[cid, sl])

    return deg_kernel, agg_kernel


def _t2_body(degp_ref, x_ref, w1_ref, hp1_ref, self1_ref, dinv_ref):
    deg = degp_ref[0, :N_NODES, 0] + degp_ref[1, :N_NODES, 0] + 1.0
    dinv = lax.rsqrt(jnp.maximum(deg, 1.0))[:, None]
    h1 = jnp.dot(x_ref[...], w1_ref[...], preferred_element_type=jnp.float32)
    hp1_ref[...] = dinv * h1
    self1_ref[...] = dinv * dinv * h1
    dinv_ref[...] = jnp.broadcast_to(dinv, (N_NODES, 16))


def _t3_body(aggp_ref, dinv_ref, self1_ref, b1_ref, w2_ref, hp2_ref, self2_ref):
    agg = aggp_ref[0, :N_NODES, :] + aggp_ref[1, :N_NODES, :]
    dinv = dinv_ref[...]
    x2 = jnp.maximum(dinv * agg + self1_ref[...] + b1_ref[...], 0.0)
    h2 = jnp.dot(x2, w2_ref[...], preferred_element_type=jnp.float32)
    hp2_ref[...] = dinv * h2
    self2_ref[...] = dinv * dinv * h2


def _t4_body(aggp_ref, dinv_ref, self2_ref, b2_ref, out_ref):
    agg = aggp_ref[0, :N_NODES, :] + aggp_ref[1, :N_NODES, :]
    out_ref[...] = dinv_ref[...] * agg + self2_ref[...] + b2_ref[...]


_t2 = pl.pallas_call(
    _t2_body,
    out_shape=(
        jax.ShapeDtypeStruct((N_NODES, D_HID), jnp.float32),
        jax.ShapeDtypeStruct((N_NODES, D_HID), jnp.float32),
        jax.ShapeDtypeStruct((N_NODES, 16), jnp.float32),
    ),
)

_t3 = pl.pallas_call(
    _t3_body,
    out_shape=(
        jax.ShapeDtypeStruct((N_NODES, N_CLS), jnp.float32),
        jax.ShapeDtypeStruct((N_NODES, N_CLS), jnp.float32),
    ),
)

_t4 = pl.pallas_call(
    _t4_body,
    out_shape=jax.ShapeDtypeStruct((N_NODES, N_CLS), jnp.float32),
)


def kernel(x, edge_index, W1, b1, W2, b2):
    src = edge_index[0].astype(jnp.int32)
    dst = edge_index[1].astype(jnp.int32)
    npad = E_PAD - N_EDGES
    pad_ids = jnp.arange(npad, dtype=jnp.int32)
    srcp = jnp.concatenate([src, pad_ids % N_NODES])
    dstp = jnp.concatenate([dst, N_NODES + pad_ids % (N_PAD - N_NODES)])
    srcp = srcp.reshape(NW, K, G)
    dstp = dstp.reshape(NW, K, G)

    ones = jnp.ones((G, 16), jnp.float32)
    zeros = jnp.zeros((N_PAD, 16), jnp.float32)

    deg_kernel, agg_kernel = _sc_kernels()
    degp = deg_kernel(dstp, ones, zeros)
    hp1, self1, dinv = _t2(degp, x, W1)
    agg1 = agg_kernel(hp1, srcp, dstp, zeros)
    hp2, self2 = _t3(agg1, dinv, self1, b1.reshape(1, 16), W2)
    agg2 = agg_kernel(hp2, srcp, dstp, zeros)
    return _t4(agg2, dinv, self2, b2.reshape(1, 16))


# speedup vs baseline: 61.8927x; 1.3072x over previous
import functools

import jax
import jax.numpy as jnp
from jax import lax
from jax.experimental import pallas as pl
from jax.experimental.pallas import tpu as pltpu
from jax.experimental.pallas import tpu_sc as plsc

N_NODES = 10000
N_EDGES = 320000
D_FEAT = 128
D_HID = 16
N_CLS = 16

NC = 2
NS = 16
NW = NC * NS
G = 128

N_PAD = 10240
ROWS_PER_TILE = N_PAD // NS
EPW = 10240
K = EPW // G
E_PAD = EPW * NW

def _worker_ids():
    cid = lax.axis_index("c")
    sid = lax.axis_index("s")
    return cid, sid, sid * NC + cid


@functools.cache
def _sc_kernels():
    mesh = plsc.VectorSubcoreMesh(core_axis_name="c", subcore_axis_name="s")
    params = pltpu.CompilerParams(use_tc_tiling_on_sc=False)

    @functools.partial(
        pl.kernel,
        mesh=mesh,
        compiler_params=params,
        out_type=jax.ShapeDtypeStruct((NC, N_PAD, 16), jnp.float32),
        scratch_types=[
            pltpu.VMEM((K, G), jnp.int32),
            pltpu.VMEM((G, 16), jnp.float32),
            pltpu.VMEM_SHARED((N_PAD, 16), jnp.float32),
        ],
    )
    def deg_kernel(dstp, ones_hbm, zeros_hbm, out, dst_v, ones_v, acc):
        cid, sid, wid = _worker_ids()
        pltpu.sync_copy(dstp.at[wid], dst_v)
        pltpu.sync_copy(ones_hbm, ones_v)
        sl = pl.ds(sid * ROWS_PER_TILE, ROWS_PER_TILE)
        pltpu.sync_copy(zeros_hbm.at[sl], acc.at[sl])
        plsc.subcore_barrier()

        def body(j, carry):
            pltpu.sync_copy(ones_v, acc.at[dst_v.at[j]], add=True)
            return carry

        lax.fori_loop(0, K, body, 0)
        plsc.subcore_barrier()
        pltpu.sync_copy(acc.at[sl], out.at[cid, sl])

    @functools.partial(
        pl.kernel,
        mesh=mesh,
        compiler_params=params,
        out_type=jax.ShapeDtypeStruct((NC, N_PAD, 16), jnp.float32),
        scratch_types=[
            pltpu.VMEM((K, G), jnp.int32),
            pltpu.VMEM((K, G), jnp.int32),
            pltpu.VMEM((G, 16), jnp.float32),
            pltpu.VMEM((G, 16), jnp.float32),
            pltpu.VMEM_SHARED((N_PAD, 16), jnp.float32),
            pltpu.SemaphoreType.DMA,
            pltpu.SemaphoreType.DMA,
        ],
    )
    def agg_kernel(hp, srcp, dstp, zeros_hbm, out,
                   src_v, dst_v, rows0, rows1, acc, sem0, sem1):
        cid, sid, wid = _worker_ids()
        pltpu.sync_copy(srcp.at[wid], src_v)
        pltpu.sync_copy(dstp.at[wid], dst_v)
        sl = pl.ds(sid * ROWS_PER_TILE, ROWS_PER_TILE)
        pltpu.sync_copy(zeros_hbm.at[sl], acc.at[sl])
        plsc.subcore_barrier()

        rows = (rows0, rows1)
        sems = (sem0, sem1)
        pltpu.async_copy(hp.at[src_v.at[0]], rows0, sem0)

        def grp(g, carry):
            j0 = g * 2
            for b in range(2):
                j = j0 + b
                jn = j + 1

                @pl.when(jn < K)
                def _prefetch():
                    pltpu.async_copy(hp.at[src_v.at[jn]], rows[1 - b], sems[1 - b])

                pltpu.make_async_copy(hp.at[pl.ds(0, G)], rows[b], sems[b]).wait()
                pltpu.sync_copy(rows[b], acc.at[dst_v.at[j]], add=True)
            return carry

        lax.fori_loop(0, K // 2, grp, 0)
        plsc.subcore_barrier()
        pltpu.sync_copy(acc.at[sl], out.at[cid, sl])

    return deg_kernel, agg_kernel


NP8 = N_PAD // 8


def _t2_body(deg2_ref, xp8_ref, w1b_ref, hp1_ref, self1_ref, dinv_ref):
    deg = deg2_ref[:NP8, :] + deg2_ref[NP8:, :] + 1.0
    dinv = lax.rsqrt(jnp.maximum(deg, 1.0))
    h1 = jnp.dot(xp8_ref[...], w1b_ref[...], preferred_element_type=jnp.float32)
    hp1_ref[...] = dinv * h1
    self1_ref[...] = dinv * dinv * h1
    dinv_ref[...] = dinv


def _t3_body(agg2_ref, dinv_ref, self1_ref, b1_ref, w2b_ref, hp2_ref, self2_ref):
    agg = agg2_ref[:NP8, :] + agg2_ref[NP8:, :]
    dinv = dinv_ref[...]
    x2 = jnp.maximum(dinv * agg + self1_ref[...] + b1_ref[...], 0.0)
    h2 = jnp.dot(x2, w2b_ref[...], preferred_element_type=jnp.float32)
    hp2_ref[...] = dinv * h2
    self2_ref[...] = dinv * dinv * h2


def _t4_body(agg2_ref, dinv_ref, self2_ref, b2_ref, out_ref):
    agg = agg2_ref[:NP8, :] + agg2_ref[NP8:, :]
    out_ref[...] = dinv_ref[...] * agg + self2_ref[...] + b2_ref[...]


_PK = jax.ShapeDtypeStruct((NP8, 128), jnp.float32)

_t2 = pl.pallas_call(_t2_body, out_shape=(_PK, _PK, _PK))
_t3 = pl.pallas_call(_t3_body, out_shape=(_PK, _PK))
_t4 = pl.pallas_call(_t4_body, out_shape=_PK)


def kernel(x, edge_index, W1, b1, W2, b2):
    src = edge_index[0].astype(jnp.int32)
    dst = edge_index[1].astype(jnp.int32)
    npad = E_PAD - N_EDGES
    pad_ids = jnp.arange(npad, dtype=jnp.int32)
    srcp = jnp.concatenate([src, pad_ids % N_NODES])
    dstp = jnp.concatenate([dst, N_NODES + pad_ids % (N_PAD - N_NODES)])
    srcp = srcp.reshape(NW, K, G)
    dstp = dstp.reshape(NW, K, G)

    ones = jnp.ones((G, 16), jnp.float32)
    zeros = jnp.zeros((N_PAD, 16), jnp.float32)

    xp8 = jnp.pad(x, ((0, N_PAD - N_NODES), (0, 0))).reshape(NP8, 8 * D_FEAT)
    eye8 = jnp.eye(8, dtype=jnp.float32)
    w1b = jnp.kron(eye8, W1)
    w2b = jnp.kron(eye8, W2)
    b1p = jnp.tile(b1, 8).reshape(1, 128)
    b2p = jnp.tile(b2, 8).reshape(1, 128)

    def sc_view(a):
        return a.reshape(2 * NP8, 128)

    def table(a):
        return a.reshape(N_PAD, 16)

    deg_kernel, agg_kernel = _sc_kernels()
    degp = deg_kernel(dstp, ones, zeros)
    hp1, self1, dinv = _t2(sc_view(degp), xp8, w1b)
    agg1 = agg_kernel(table(hp1), srcp, dstp, zeros)
    hp2, self2 = _t3(sc_view(agg1), dinv, self1, b1p, w2b)
    agg2 = agg_kernel(table(hp2), srcp, dstp, zeros)
    outp = _t4(sc_view(agg2), dinv, self2, b2p)
    return outp.reshape(N_PAD, 16)[:N_NODES]


# speedup vs baseline: 67.6253x; 1.0926x over previous
import functools

import jax
import jax.numpy as jnp
from jax import lax
from jax.experimental import pallas as pl
from jax.experimental.pallas import tpu as pltpu
from jax.experimental.pallas import tpu_sc as plsc

N_NODES = 10000
N_EDGES = 320000
D_FEAT = 128
D_HID = 16
N_CLS = 16

NC = 2
NS = 16
NW = NC * NS
G = 128

N_PAD = 10240
ROWS_PER_TILE = N_PAD // NS
EPW = 10240
K = EPW // G
E_PAD = EPW * NW

def _worker_ids():
    cid = lax.axis_index("c")
    sid = lax.axis_index("s")
    return cid, sid, sid * NC + cid


@functools.cache
def _sc_kernels():
    mesh = plsc.VectorSubcoreMesh(core_axis_name="c", subcore_axis_name="s")
    params = pltpu.CompilerParams(use_tc_tiling_on_sc=False)

    @functools.partial(
        pl.kernel,
        mesh=mesh,
        compiler_params=params,
        out_type=jax.ShapeDtypeStruct((NC, N_PAD, 16), jnp.float32),
        scratch_types=[
            pltpu.VMEM((K, G), jnp.int32),
            pltpu.VMEM((G, 16), jnp.float32),
            pltpu.VMEM_SHARED((N_PAD, 16), jnp.float32),
            pltpu.SemaphoreType.DMA,
        ],
    )
    def deg_kernel(dstp, ones_hbm, zeros_hbm, out, dst_v, ones_v, acc, sem):
        cid, sid, wid = _worker_ids()
        pltpu.sync_copy(dstp.at[wid], dst_v)
        pltpu.sync_copy(ones_hbm, ones_v)
        sl = pl.ds(sid * ROWS_PER_TILE, ROWS_PER_TILE)
        pltpu.sync_copy(zeros_hbm.at[sl], acc.at[sl])
        plsc.subcore_barrier()

        def fire(j, carry):
            pltpu.async_copy(ones_v, acc.at[dst_v.at[j]], sem, add=True)
            return carry

        lax.fori_loop(0, K, fire, 0)

        def drain(j, carry):
            pltpu.make_async_copy(ones_hbm, ones_v, sem).wait()
            return carry

        lax.fori_loop(0, K, drain, 0)
        plsc.subcore_barrier()
        pltpu.sync_copy(acc.at[sl], out.at[cid, sl])

    @functools.partial(
        pl.kernel,
        mesh=mesh,
        compiler_params=params,
        out_type=jax.ShapeDtypeStruct((NC, N_PAD, 16), jnp.float32),
        scratch_types=[
            pltpu.VMEM((K, G), jnp.int32),
            pltpu.VMEM((K, G), jnp.int32),
            pltpu.VMEM((G, 16), jnp.float32),
            pltpu.VMEM((G, 16), jnp.float32),
            pltpu.VMEM((G, 16), jnp.float32),
            pltpu.VMEM((G, 16), jnp.float32),
            pltpu.VMEM_SHARED((N_PAD, 16), jnp.float32),
            pltpu.SemaphoreType.DMA,
            pltpu.SemaphoreType.DMA,
            pltpu.SemaphoreType.DMA,
            pltpu.SemaphoreType.DMA,
            pltpu.SemaphoreType.DMA,
            pltpu.SemaphoreType.DMA,
            pltpu.SemaphoreType.DMA,
            pltpu.SemaphoreType.DMA,
        ],
    )
    def agg_kernel(hp, srcp, dstp, zeros_hbm, out,
                   src_v, dst_v, rows0, rows1, rows2, rows3, acc,
                   gs0, gs1, gs2, gs3, ss0, ss1, ss2, ss3):
        cid, sid, wid = _worker_ids()
        pltpu.sync_copy(srcp.at[wid], src_v)
        pltpu.sync_copy(dstp.at[wid], dst_v)
        sl = pl.ds(sid * ROWS_PER_TILE, ROWS_PER_TILE)
        pltpu.sync_copy(zeros_hbm.at[sl], acc.at[sl])
        plsc.subcore_barrier()

        rows = (rows0, rows1, rows2, rows3)
        gs = (gs0, gs1, gs2, gs3)
        ss = (ss0, ss1, ss2, ss3)

        pltpu.async_copy(hp.at[src_v.at[0]], rows0, gs0)
        pltpu.async_copy(hp.at[src_v.at[1]], rows1, gs1)

        def grp(g, carry):
            j0 = g * 4
            for b in range(4):
                j = j0 + b
                pltpu.make_async_copy(hp.at[pl.ds(0, G)], rows[b], gs[b]).wait()
                pltpu.async_copy(rows[b], acc.at[dst_v.at[j]], ss[b], add=True)
                jn = j + 2
                bn = (b + 2) % 4

                @pl.when(jn < K)
                def _prefetch():
                    @pl.when(j >= 2)
                    def _wait_scatter():
                        pltpu.make_async_copy(
                            hp.at[pl.ds(0, G)], rows[bn], ss[bn]).wait()

                    pltpu.async_copy(hp.at[src_v.at[jn]], rows[bn], gs[bn])
            return carry

        lax.fori_loop(0, K // 4, grp, 0)
        for b in range(4):
            pltpu.make_async_copy(hp.at[pl.ds(0, G)], rows[b], ss[b]).wait()
        plsc.subcore_barrier()
        pltpu.sync_copy(acc.at[sl], out.at[cid, sl])

    return deg_kernel, agg_kernel


NP8 = N_PAD // 8


def _t2_body(deg2_ref, xp8_ref, w1b_ref, hp1_ref, self1_ref, dinv_ref):
    deg = deg2_ref[:NP8, :] + deg2_ref[NP8:, :] + 1.0
    dinv = lax.rsqrt(jnp.maximum(deg, 1.0))
    h1 = jnp.dot(xp8_ref[...], w1b_ref[...], preferred_element_type=jnp.float32)
    hp1_ref[...] = dinv * h1
    self1_ref[...] = dinv * dinv * h1
    dinv_ref[...] = dinv


def _t3_body(agg2_ref, dinv_ref, self1_ref, b1_ref, w2b_ref, hp2_ref, self2_ref):
    agg = agg2_ref[:NP8, :] + agg2_ref[NP8:, :]
    dinv = dinv_ref[...]
    x2 = jnp.maximum(dinv * agg + self1_ref[...] + b1_ref[...], 0.0)
    h2 = jnp.dot(x2, w2b_ref[...], preferred_element_type=jnp.float32)
    hp2_ref[...] = dinv * h2
    self2_ref[...] = dinv * dinv * h2


def _t4_body(agg2_ref, dinv_ref, self2_ref, b2_ref, out_ref):
    agg = agg2_ref[:NP8, :] + agg2_ref[NP8:, :]
    out_ref[...] = dinv_ref[...] * agg + self2_ref[...] + b2_ref[...]


_PK = jax.ShapeDtypeStruct((NP8, 128), jnp.float32)

_t2 = pl.pallas_call(_t2_body, out_shape=(_PK, _PK, _PK))
_t3 = pl.pallas_call(_t3_body, out_shape=(_PK, _PK))
_t4 = pl.pallas_call(_t4_body, out_shape=_PK)


def kernel(x, edge_index, W1, b1, W2, b2):
    src = edge_index[0].astype(jnp.int32)
    dst = edge_index[1].astype(jnp.int32)
    npad = E_PAD - N_EDGES
    pad_ids = jnp.arange(npad, dtype=jnp.int32)
    srcp = jnp.concatenate([src, pad_ids % N_NODES])
    dstp = jnp.concatenate([dst, N_NODES + pad_ids % (N_PAD - N_NODES)])
    srcp = srcp.reshape(NW, K, G)
    dstp = dstp.reshape(NW, K, G)

    ones = jnp.ones((G, 16), jnp.float32)
    zeros = jnp.zeros((N_PAD, 16), jnp.float32)

    xp8 = jnp.pad(x, ((0, N_PAD - N_NODES), (0, 0))).reshape(NP8, 8 * D_FEAT)
    eye8 = jnp.eye(8, dtype=jnp.float32)
    w1b = jnp.kron(eye8, W1)
    w2b = jnp.kron(eye8, W2)
    b1p = jnp.tile(b1, 8).reshape(1, 128)
    b2p = jnp.tile(b2, 8).reshape(1, 128)

    def sc_view(a):
        return a.reshape(2 * NP8, 128)

    def table(a):
        return a.reshape(N_PAD, 16)

    deg_kernel, agg_kernel = _sc_kernels()
    degp = deg_kernel(dstp, ones, zeros)
    hp1, self1, dinv = _t2(sc_view(degp), xp8, w1b)
    agg1 = agg_kernel(table(hp1), srcp, dstp, zeros)
    hp2, self2 = _t3(sc_view(agg1), dinv, self1, b1p, w2b)
    agg2 = agg_kernel(table(hp2), srcp, dstp, zeros)
    outp = _t4(sc_view(agg2), dinv, self2, b2p)
    return outp.reshape(N_PAD, 16)[:N_NODES]


# speedup vs baseline: 79.8568x; 1.1809x over previous
import functools

import jax
import jax.numpy as jnp
from jax import lax
from jax.experimental import pallas as pl
from jax.experimental.pallas import tpu as pltpu
from jax.experimental.pallas import tpu_sc as plsc

N_NODES = 10000
N_EDGES = 320000
D_FEAT = 128
D_HID = 16
N_CLS = 16

NC = 2
NS = 16
NW = NC * NS
G = 128

N_PAD = 10240
ROWS_PER_TILE = N_PAD // NS
EPW = 10240
K = EPW // G
E_PAD = EPW * NW

def _worker_ids():
    cid = lax.axis_index("c")
    sid = lax.axis_index("s")
    return cid, sid, sid * NC + cid


@functools.cache
def _sc_kernels():
    mesh = plsc.VectorSubcoreMesh(core_axis_name="c", subcore_axis_name="s")
    params = pltpu.CompilerParams(use_tc_tiling_on_sc=False)

    @functools.partial(
        pl.kernel,
        mesh=mesh,
        compiler_params=params,
        out_type=jax.ShapeDtypeStruct((NC, N_PAD, 16), jnp.float32),
        scratch_types=[
            pltpu.VMEM((K, G), jnp.int32),
            pltpu.VMEM((G, 16), jnp.float32),
            pltpu.VMEM_SHARED((N_PAD, 16), jnp.float32),
            pltpu.SemaphoreType.DMA,
        ],
    )
    def deg_kernel(dstp, ones_hbm, zeros_hbm, out, dst_v, ones_v, acc, sem):
        cid, sid, wid = _worker_ids()
        pltpu.sync_copy(dstp.at[wid], dst_v)
        pltpu.sync_copy(ones_hbm, ones_v)
        sl = pl.ds(sid * ROWS_PER_TILE, ROWS_PER_TILE)
        pltpu.sync_copy(zeros_hbm.at[sl], acc.at[sl])
        plsc.subcore_barrier()

        def fire(j, carry):
            pltpu.async_copy(ones_v, acc.at[dst_v.at[j]], sem, add=True)
            return carry

        lax.fori_loop(0, K, fire, 0)

        def drain(j, carry):
            pltpu.make_async_copy(ones_hbm, ones_v, sem).wait()
            return carry

        lax.fori_loop(0, K, drain, 0)
        plsc.subcore_barrier()
        pltpu.sync_copy(acc.at[sl], out.at[cid, sl])

    @functools.partial(
        pl.kernel,
        mesh=mesh,
        compiler_params=params,
        out_type=jax.ShapeDtypeStruct((NC, N_PAD, 16), jnp.float32),
        scratch_types=[
            pltpu.VMEM((K, G), jnp.int32),
            pltpu.VMEM((K, G), jnp.int32),
            pltpu.VMEM((2 * G, 16), jnp.float32),
            pltpu.VMEM((2 * G, 16), jnp.float32),
            pltpu.VMEM((2 * G, 16), jnp.float32),
            pltpu.VMEM((2 * G, 16), jnp.float32),
            pltpu.VMEM_SHARED((N_PAD, 16), jnp.float32),
            pltpu.SemaphoreType.DMA,
            pltpu.SemaphoreType.DMA,
            pltpu.SemaphoreType.DMA,
            pltpu.SemaphoreType.DMA,
            pltpu.SemaphoreType.DMA,
            pltpu.SemaphoreType.DMA,
            pltpu.SemaphoreType.DMA,
            pltpu.SemaphoreType.DMA,
        ],
    )
    def agg_kernel(hp, srcp, dstp, zeros_hbm, out,
                   src_v, dst_v, rows0, rows1, rows2, rows3, acc,
                   gs0, gs1, gs2, gs3, ss0, ss1, ss2, ss3):
        cid, sid, wid = _worker_ids()
        pltpu.sync_copy(srcp.at[wid], src_v)
        pltpu.sync_copy(dstp.at[wid], dst_v)
        sl = pl.ds(sid * ROWS_PER_TILE, ROWS_PER_TILE)
        pltpu.sync_copy(zeros_hbm.at[sl], acc.at[sl])
        plsc.subcore_barrier()

        rows = (rows0, rows1, rows2, rows3)
        gs = (gs0, gs1, gs2, gs3)
        ss = (ss0, ss1, ss2, ss3)
        NI = K // 2

        def gather2(i, b):
            pltpu.async_copy(hp.at[src_v.at[2 * i]], rows[b].at[pl.ds(0, G)], gs[b])
            pltpu.async_copy(
                hp.at[src_v.at[2 * i + 1]], rows[b].at[pl.ds(G, G)], gs[b])

        gather2(0, 0)
        gather2(1, 1)

        def grp(g, carry):
            i0 = g * 4
            for b in range(4):
                i = i0 + b
                pltpu.make_async_copy(
                    hp.at[pl.ds(0, 2 * G)], rows[b], gs[b]).wait()
                pltpu.async_copy(
                    rows[b].at[pl.ds(0, G)], acc.at[dst_v.at[2 * i]],
                    ss[b], add=True)
                pltpu.async_copy(
                    rows[b].at[pl.ds(G, G)], acc.at[dst_v.at[2 * i + 1]],
                    ss[b], add=True)
                inext = i + 2
                bn = (b + 2) % 4

                @pl.when(inext < NI)
                def _prefetch():
                    @pl.when(i >= 2)
                    def _wait_scatter():
                        pltpu.make_async_copy(
                            hp.at[pl.ds(0, 2 * G)], rows[bn], ss[bn]).wait()

                    gather2(inext, bn)
            return carry

        lax.fori_loop(0, NI // 4, grp, 0)
        for b in range(4):
            pltpu.make_async_copy(hp.at[pl.ds(0, 2 * G)], rows[b], ss[b]).wait()
        plsc.subcore_barrier()
        pltpu.sync_copy(acc.at[sl], out.at[cid, sl])

    return deg_kernel, agg_kernel


NP8 = N_PAD // 8


def _t2_body(deg2_ref, xp8_ref, w1b_ref, hp1_ref, self1_ref, dinv_ref):
    deg = deg2_ref[:NP8, :] + deg2_ref[NP8:, :] + 1.0
    dinv = lax.rsqrt(jnp.maximum(deg, 1.0))
    h1 = jnp.dot(xp8_ref[...], w1b_ref[...], preferred_element_type=jnp.float32)
    hp1_ref[...] = dinv * h1
    self1_ref[...] = dinv * dinv * h1
    dinv_ref[...] = dinv


def _t3_body(agg2_ref, dinv_ref, self1_ref, b1_ref, w2b_ref, hp2_ref, self2_ref):
    agg = agg2_ref[:NP8, :] + agg2_ref[NP8:, :]
    dinv = dinv_ref[...]
    x2 = jnp.maximum(dinv * agg + self1_ref[...] + b1_ref[...], 0.0)
    h2 = jnp.dot(x2, w2b_ref[...], preferred_element_type=jnp.float32)
    hp2_ref[...] = dinv * h2
    self2_ref[...] = dinv * dinv * h2


def _t4_body(agg2_ref, dinv_ref, self2_ref, b2_ref, out_ref):
    agg = agg2_ref[:NP8, :] + agg2_ref[NP8:, :]
    out_ref[...] = dinv_ref[...] * agg + self2_ref[...] + b2_ref[...]


_PK = jax.ShapeDtypeStruct((NP8, 128), jnp.float32)

_t2 = pl.pallas_call(_t2_body, out_shape=(_PK, _PK, _PK))
_t3 = pl.pallas_call(_t3_body, out_shape=(_PK, _PK))
_t4 = pl.pallas_call(_t4_body, out_shape=_PK)


def kernel(x, edge_index, W1, b1, W2, b2):
    src = edge_index[0].astype(jnp.int32)
    dst = edge_index[1].astype(jnp.int32)
    npad = E_PAD - N_EDGES
    pad_ids = jnp.arange(npad, dtype=jnp.int32)
    srcp = jnp.concatenate([src, pad_ids % N_NODES])
    dstp = jnp.concatenate([dst, N_NODES + pad_ids % (N_PAD - N_NODES)])
    srcp = srcp.reshape(NW, K, G)
    dstp = dstp.reshape(NW, K, G)

    ones = jnp.ones((G, 16), jnp.float32)
    zeros = jnp.zeros((N_PAD, 16), jnp.float32)

    xp8 = jnp.pad(x, ((0, N_PAD - N_NODES), (0, 0))).reshape(NP8, 8 * D_FEAT)
    eye8 = jnp.eye(8, dtype=jnp.float32)
    w1b = jnp.kron(eye8, W1)
    w2b = jnp.kron(eye8, W2)
    b1p = jnp.tile(b1, 8).reshape(1, 128)
    b2p = jnp.tile(b2, 8).reshape(1, 128)

    def sc_view(a):
        return a.reshape(2 * NP8, 128)

    def table(a):
        return a.reshape(N_PAD, 16)

    deg_kernel, agg_kernel = _sc_kernels()
    degp = deg_kernel(dstp, ones, zeros)
    hp1, self1, dinv = _t2(sc_view(degp), xp8, w1b)
    agg1 = agg_kernel(table(hp1), srcp, dstp, zeros)
    hp2, self2 = _t3(sc_view(agg1), dinv, self1, b1p, w2b)
    agg2 = agg_kernel(table(hp2), srcp, dstp, zeros)
    outp = _t4(sc_view(agg2), dinv, self2, b2p)
    return outp.reshape(N_PAD, 16)[:N_NODES]


# speedup vs baseline: 86.9330x; 1.0886x over previous
import functools

import jax
import jax.numpy as jnp
from jax import lax
from jax.experimental import pallas as pl
from jax.experimental.pallas import tpu as pltpu
from jax.experimental.pallas import tpu_sc as plsc

N_NODES = 10000
N_EDGES = 320000
D_FEAT = 128
D_HID = 16
N_CLS = 16

NC = 2
NS = 16
NW = NC * NS
G = 128

N_PAD = 10240
ROWS_PER_TILE = N_PAD // NS
EPW = 10240
K = EPW // G
E_PAD = EPW * NW

def _worker_ids():
    cid = lax.axis_index("c")
    sid = lax.axis_index("s")
    return cid, sid, sid * NC + cid


@functools.cache
def _sc_kernels():
    mesh = plsc.VectorSubcoreMesh(core_axis_name="c", subcore_axis_name="s")
    params = pltpu.CompilerParams(use_tc_tiling_on_sc=False)

    @functools.partial(
        pl.kernel,
        mesh=mesh,
        compiler_params=params,
        out_type=jax.ShapeDtypeStruct((NC, N_PAD, 16), jnp.float32),
        scratch_types=[
            pltpu.VMEM((K, G), jnp.int32),
            pltpu.VMEM((G, 16), jnp.float32),
            pltpu.VMEM_SHARED((N_PAD, 16), jnp.float32),
            pltpu.SemaphoreType.DMA,
        ],
    )
    def deg_kernel(dstp, ones_hbm, zeros_hbm, out, dst_v, ones_v, acc, sem):
        cid, sid, wid = _worker_ids()
        pltpu.sync_copy(dstp.at[wid], dst_v)
        pltpu.sync_copy(ones_hbm, ones_v)
        sl = pl.ds(sid * ROWS_PER_TILE, ROWS_PER_TILE)
        pltpu.sync_copy(zeros_hbm.at[sl], acc.at[sl])
        plsc.subcore_barrier()

        def fire(j, carry):
            pltpu.async_copy(ones_v, acc.at[dst_v.at[j]], sem, add=True)
            return carry

        lax.fori_loop(0, K, fire, 0)

        def drain(j, carry):
            pltpu.make_async_copy(dstp.at[wid], dst_v, sem).wait()
            return carry

        lax.fori_loop(0, (K * G * 16 * 4) // (K * G * 4), drain, 0)
        plsc.subcore_barrier()
        pltpu.sync_copy(acc.at[sl], out.at[cid, sl])

    @functools.partial(
        pl.kernel,
        mesh=mesh,
        compiler_params=params,
        out_type=jax.ShapeDtypeStruct((NC, N_PAD, 16), jnp.float32),
        scratch_types=[
            pltpu.VMEM((K, G), jnp.int32),
            pltpu.VMEM((K, G), jnp.int32),
            pltpu.VMEM((4 * G, 16), jnp.float32),
            pltpu.VMEM((4 * G, 16), jnp.float32),
            pltpu.VMEM((4 * G, 16), jnp.float32),
            pltpu.VMEM((4 * G, 16), jnp.float32),
            pltpu.VMEM_SHARED((N_PAD, 16), jnp.float32),
            pltpu.SemaphoreType.DMA,
            pltpu.SemaphoreType.DMA,
            pltpu.SemaphoreType.DMA,
            pltpu.SemaphoreType.DMA,
            pltpu.SemaphoreType.DMA,
            pltpu.SemaphoreType.DMA,
            pltpu.SemaphoreType.DMA,
            pltpu.SemaphoreType.DMA,
        ],
    )
    def agg_kernel(hp, srcp, dstp, zeros_hbm, out,
                   src_v, dst_v, rows0, rows1, rows2, rows3, acc,
                   gs0, gs1, gs2, gs3, ss0, ss1, ss2, ss3):
        cid, sid, wid = _worker_ids()
        pltpu.sync_copy(srcp.at[wid], src_v)
        pltpu.sync_copy(dstp.at[wid], dst_v)
        sl = pl.ds(sid * ROWS_PER_TILE, ROWS_PER_TILE)
        pltpu.sync_copy(zeros_hbm.at[sl], acc.at[sl])
        plsc.subcore_barrier()

        rows = (rows0, rows1, rows2, rows3)
        gs = (gs0, gs1, gs2, gs3)
        ss = (ss0, ss1, ss2, ss3)
        S = 4
        NI = K // S

        def gatherS(i, b):
            for r in range(S):
                pltpu.async_copy(
                    hp.at[src_v.at[S * i + r]], rows[b].at[pl.ds(r * G, G)],
                    gs[b])

        gatherS(0, 0)
        gatherS(1, 1)

        def grp(g, carry):
            i0 = g * 4
            for b in range(4):
                i = i0 + b
                pltpu.make_async_copy(
                    hp.at[pl.ds(0, S * G)], rows[b], gs[b]).wait()
                for r in range(S):
                    pltpu.async_copy(
                        rows[b].at[pl.ds(r * G, G)],
                        acc.at[dst_v.at[S * i + r]], ss[b], add=True)
                inext = i + 2
                bn = (b + 2) % 4

                @pl.when(inext < NI)
                def _prefetch():
                    @pl.when(i >= 2)
                    def _wait_scatter():
                        pltpu.make_async_copy(
                            hp.at[pl.ds(0, S * G)], rows[bn], ss[bn]).wait()

                    gatherS(inext, bn)
            return carry

        lax.fori_loop(0, NI // 4, grp, 0)
        for b in range(4):
            pltpu.make_async_copy(hp.at[pl.ds(0, S * G)], rows[b], ss[b]).wait()
        plsc.subcore_barrier()
        pltpu.sync_copy(acc.at[sl], out.at[cid, sl])

    return deg_kernel, agg_kernel


NP8 = N_PAD // 8


def _t2_body(deg2_ref, xp8_ref, w1b_ref, hp1_ref, self1_ref, dinv_ref):
    deg = deg2_ref[:NP8, :] + deg2_ref[NP8:, :] + 1.0
    dinv = lax.rsqrt(jnp.maximum(deg, 1.0))
    h1 = jnp.dot(xp8_ref[...], w1b_ref[...], preferred_element_type=jnp.float32)
    hp1_ref[...] = dinv * h1
    self1_ref[...] = dinv * dinv * h1
    dinv_ref[...] = dinv


def _t3_body(agg2_ref, dinv_ref, self1_ref, b1_ref, w2b_ref, hp2_ref, self2_ref):
    agg = agg2_ref[:NP8, :] + agg2_ref[NP8:, :]
    dinv = dinv_ref[...]
    x2 = jnp.maximum(dinv * agg + self1_ref[...] + b1_ref[...], 0.0)
    h2 = jnp.dot(x2, w2b_ref[...], preferred_element_type=jnp.float32)
    hp2_ref[...] = dinv * h2
    self2_ref[...] = dinv * dinv * h2


def _t4_body(agg2_ref, dinv_ref, self2_ref, b2_ref, out_ref):
    agg = agg2_ref[:NP8, :] + agg2_ref[NP8:, :]
    out_ref[...] = dinv_ref[...] * agg + self2_ref[...] + b2_ref[...]


_PK = jax.ShapeDtypeStruct((NP8, 128), jnp.float32)

_t2 = pl.pallas_call(_t2_body, out_shape=(_PK, _PK, _PK))
_t3 = pl.pallas_call(_t3_body, out_shape=(_PK, _PK))
_t4 = pl.pallas_call(_t4_body, out_shape=_PK)


def kernel(x, edge_index, W1, b1, W2, b2):
    src = edge_index[0].astype(jnp.int32)
    dst = edge_index[1].astype(jnp.int32)
    npad = E_PAD - N_EDGES
    pad_ids = jnp.arange(npad, dtype=jnp.int32)
    srcp = jnp.concatenate([src, pad_ids % N_NODES])
    dstp = jnp.concatenate([dst, N_NODES + pad_ids % (N_PAD - N_NODES)])
    srcp = srcp.reshape(NW, K, G)
    dstp = dstp.reshape(NW, K, G)

    ones = jnp.ones((G, 16), jnp.float32)
    zeros = jnp.zeros((N_PAD, 16), jnp.float32)

    xp8 = jnp.pad(x, ((0, N_PAD - N_NODES), (0, 0))).reshape(NP8, 8 * D_FEAT)
    eye8 = jnp.eye(8, dtype=jnp.float32)
    w1b = jnp.kron(eye8, W1)
    w2b = jnp.kron(eye8, W2)
    b1p = jnp.tile(b1, 8).reshape(1, 128)
    b2p = jnp.tile(b2, 8).reshape(1, 128)

    def sc_view(a):
        return a.reshape(2 * NP8, 128)

    def table(a):
        return a.reshape(N_PAD, 16)

    deg_kernel, agg_kernel = _sc_kernels()
    degp = deg_kernel(dstp, ones, zeros)
    hp1, self1, dinv = _t2(sc_view(degp), xp8, w1b)
    agg1 = agg_kernel(table(hp1), srcp, dstp, zeros)
    hp2, self2 = _t3(sc_view(agg1), dinv, self1, b1p, w2b)
    agg2 = agg_kernel(table(hp2), srcp, dstp, zeros)
    outp = _t4(sc_view(agg2), dinv, self2, b2p)
    return outp.reshape(N_PAD, 16)[:N_NODES]
